# baseline (speedup 1.0000x reference)
import functools

import jax
import jax.numpy as jnp
import numpy as np
from jax import lax
from jax.experimental import pallas as pl
from jax.experimental.pallas import tpu as pltpu
from jax.experimental.pallas import tpu_sc as plsc

_N = 131072
_K = 256
_NB = 32
_NE = _NB + 1
_L = 16
_NC = 2
_NS = 16
_TILES = _NC * _NS
_ROWS_PER_TILE = _N // _TILES
_CHUNK = 128
_NCHUNKS = _ROWS_PER_TILE // _CHUNK
_GROUPS = _K // _L
_CHUNK_ELEMS = _CHUNK * _K


_MM_BLK = 16384
_MM_GRID = _N // _MM_BLK


def _minmax_body(x_ref, mn_ref, mx_ref, params_ref):
    i = pl.program_id(0)
    m = jnp.min(x_ref[...])
    mm = jnp.max(x_ref[...])

    @pl.when(i == 0)
    def _():
        mn_ref[0, 0] = m
        mx_ref[0, 0] = mm

    @pl.when(i != 0)
    def _():
        mn_ref[0, 0] = jnp.minimum(mn_ref[0, 0], m)
        mx_ref[0, 0] = jnp.maximum(mx_ref[0, 0], mm)

    @pl.when(i == _MM_GRID - 1)
    def _():
        r0 = mn_ref[0, 0]
        r1 = mx_ref[0, 0]
        scale = jnp.where(r1 > r0, _NB / (r1 - r0), 0.0).astype(jnp.float32)
        params_ref[0, :] = jnp.full((_L,), r0, jnp.float32)
        params_ref[1, :] = jnp.full((_L,), scale, jnp.float32)


def _minmax(x):
    return pl.pallas_call(
        _minmax_body,
        grid=(_MM_GRID,),
        in_specs=[pl.BlockSpec((_MM_BLK, _K), lambda i: (i, 0))],
        out_specs=[
            pl.BlockSpec(memory_space=pltpu.SMEM),
            pl.BlockSpec(memory_space=pltpu.SMEM),
            pl.BlockSpec((2, _L), lambda i: (0, 0)),
        ],
        out_shape=[
            jax.ShapeDtypeStruct((1, 1), jnp.float32),
            jax.ShapeDtypeStruct((1, 1), jnp.float32),
            jax.ShapeDtypeStruct((2, _L), jnp.float32),
        ],
    )(x)


_sc_mesh = plsc.VectorSubcoreMesh(core_axis_name="c", subcore_axis_name="s")


@functools.partial(
    pl.kernel,
    mesh=_sc_mesh,
    compiler_params=pltpu.CompilerParams(needs_layout_passes=False, use_tc_tiling_on_sc=False),
    out_type=jax.ShapeDtypeStruct((_TILES, _NE, _K), jnp.float32),
    scratch_types=[
        pltpu.VMEM((_CHUNK, _K), jnp.float32),
        pltpu.VMEM((_CHUNK, _K), jnp.float32),
        pltpu.VMEM((2, _L), jnp.float32),
        pltpu.VMEM((_NE, _K), jnp.float32),
        pltpu.SemaphoreType.DMA,
        pltpu.SemaphoreType.DMA,
    ],
)
def _hist(mean_hbm, params_hbm, out_hbm, chunk0, chunk1, params_v, acc_v,
          sem0, sem1):
    c = lax.axis_index("c")
    s = lax.axis_index("s")
    wid = s * _NC + c

    pltpu.sync_copy(params_hbm, params_v)
    r0v = params_v[0, :]
    sv = params_v[1, :]

    zero = jnp.zeros((_L,), jnp.float32)
    ones = jnp.ones((_L,), jnp.float32)
    lane = lax.iota(jnp.int32, _L)

    def zrow(j, carry):
        for g in range(_GROUPS):
            acc_v[j, pl.ds(g * _L, _L)] = zero
        return carry

    lax.fori_loop(0, _NE, zrow, None)

    def copy(ci, buf, sem):
        row0 = wid * _ROWS_PER_TILE + ci * _CHUNK
        return pltpu.make_async_copy(
            mean_hbm.at[pl.ds(row0, _CHUNK)], buf, sem)

    def process(buf):
        @plsc.parallel_loop(0, _CHUNK * _GROUPS, unroll=16)
        def _(v):
            r = v // _GROUPS
            g = v % _GROUPS
            x = buf[r, pl.ds(g * _L, _L)]
            t = (x - r0v) * sv
            b = t.astype(jnp.int32)
            col = lane + g * _L
            plsc.addupdate_scatter(acc_v, [b, col], ones)

    copy(0, chunk0, sem0).start()
    copy(1, chunk1, sem1).start()

    def outer(ci2, carry):
        for b_, (buf, sem) in enumerate(((chunk0, sem0), (chunk1, sem1))):
            ci = ci2 * 2 + b_
            copy(ci, buf, sem).wait()
            process(buf)

            @pl.when(ci + 2 < _NCHUNKS)
            def _():
                copy(ci + 2, buf, sem).start()

        return carry

    lax.fori_loop(0, _NCHUNKS // 2, outer, None)
    pltpu.sync_copy(acc_v, out_hbm.at[wid])


_INV_LN2 = np.float32(1.0 / np.log(2.0))


def _entropy_body(parts_ref, pi_ref, mn_ref, mx_ref, out_ref):
    counts = jnp.sum(parts_ref[...], axis=0)
    counts = counts[:_NB, :]
    d = counts * np.float32(1.0 / _N)
    ent = jnp.sum(-d * jnp.log(d + np.float32(1e-7)), axis=0, keepdims=True)
    ixt = jnp.sum((1.0 - pi_ref[...]) * ent) * _INV_LN2
    out_ref[0, 0] = jnp.where(mx_ref[0, 0] > mn_ref[0, 0], ixt, 0.0)


def _entropy(parts, pi, mn, mx):
    return pl.pallas_call(
        _entropy_body,
        in_specs=[
            pl.BlockSpec((_TILES, _NE, _K), lambda: (0, 0, 0)),
            pl.BlockSpec((1, _K), lambda: (0, 0)),
            pl.BlockSpec(memory_space=pltpu.SMEM),
            pl.BlockSpec(memory_space=pltpu.SMEM),
        ],
        out_specs=pl.BlockSpec(memory_space=pltpu.SMEM),
        out_shape=jax.ShapeDtypeStruct((1, 1), jnp.float32),
    )(parts, pi, mn, mx)


def kernel(mean_t, pi):
    mn, mx, params = _minmax(mean_t)
    parts = _hist(mean_t, params)
    return _entropy(parts, pi, mn, mx)[0]


# speedup vs baseline: 1.6051x; 1.6051x over previous
import functools

import jax
import jax.numpy as jnp
import numpy as np
from jax import lax
from jax.experimental import pallas as pl
from jax.experimental.pallas import tpu as pltpu
from jax.experimental.pallas import tpu_sc as plsc

_N = 131072
_K = 256
_NB = 32
_NE = _NB + 1
_L = 16
_NC = 2
_NS = 16
_TILES = _NC * _NS
_ROWS_PER_TILE = _N // _TILES
_CHUNK = 128
_NCHUNKS = _ROWS_PER_TILE // _CHUNK
_GROUPS = _K // _L
_CHUNK_ELEMS = _CHUNK * _K


_MM_BLK = 16384
_MM_GRID = _N // _MM_BLK


def _minmax_body(x_ref, mn_ref, mx_ref, params_ref):
    i = pl.program_id(0)
    m = jnp.min(x_ref[...])
    mm = jnp.max(x_ref[...])

    @pl.when(i == 0)
    def _():
        mn_ref[0, 0] = m
        mx_ref[0, 0] = mm

    @pl.when(i != 0)
    def _():
        mn_ref[0, 0] = jnp.minimum(mn_ref[0, 0], m)
        mx_ref[0, 0] = jnp.maximum(mx_ref[0, 0], mm)

    @pl.when(i == _MM_GRID - 1)
    def _():
        r0 = mn_ref[0, 0]
        r1 = mx_ref[0, 0]
        scale = jnp.where(r1 > r0, _NB / (r1 - r0), 0.0).astype(jnp.float32)
        params_ref[0, :] = jnp.full((_L,), r0, jnp.float32)
        params_ref[1, :] = jnp.full((_L,), scale, jnp.float32)


def _minmax(x):
    return pl.pallas_call(
        _minmax_body,
        grid=(_MM_GRID,),
        in_specs=[pl.BlockSpec((_MM_BLK, _K), lambda i: (i, 0))],
        out_specs=[
            pl.BlockSpec(memory_space=pltpu.SMEM),
            pl.BlockSpec(memory_space=pltpu.SMEM),
            pl.BlockSpec((2, _L), lambda i: (0, 0)),
        ],
        out_shape=[
            jax.ShapeDtypeStruct((1, 1), jnp.float32),
            jax.ShapeDtypeStruct((1, 1), jnp.float32),
            jax.ShapeDtypeStruct((2, _L), jnp.float32),
        ],
    )(x)


_sc_mesh = plsc.VectorSubcoreMesh(core_axis_name="c", subcore_axis_name="s")


@functools.partial(
    pl.kernel,
    mesh=_sc_mesh,
    compiler_params=pltpu.CompilerParams(needs_layout_passes=False),
    out_type=jax.ShapeDtypeStruct((_TILES, _NE * _K), jnp.float32),
    scratch_types=[
        pltpu.VMEM((_CHUNK, _K), jnp.float32),
        pltpu.VMEM((_CHUNK, _K), jnp.float32),
        pltpu.VMEM((2, _L), jnp.float32),
        pltpu.VMEM((_NE * _K,), jnp.float32),
        pltpu.SemaphoreType.DMA,
        pltpu.SemaphoreType.DMA,
    ],
)
def _hist(mean_hbm, params_hbm, out_hbm, chunk0, chunk1, params_v, acc_v,
          sem0, sem1):
    c = lax.axis_index("c")
    s = lax.axis_index("s")
    wid = s * _NC + c

    pltpu.sync_copy(params_hbm, params_v)
    r0v = params_v[0, :]
    sv = params_v[1, :]

    zero = jnp.zeros((_L,), jnp.float32)
    ones = jnp.ones((_L,), jnp.float32)
    lane = lax.iota(jnp.int32, _L)

    def zrow(j, carry):
        for g in range(_GROUPS):
            acc_v[pl.ds(j * _K + g * _L, _L)] = zero
        return carry

    lax.fori_loop(0, _NE, zrow, None)

    def copy(ci, buf, sem):
        row0 = wid * _ROWS_PER_TILE + ci * _CHUNK
        return pltpu.make_async_copy(
            mean_hbm.at[pl.ds(row0, _CHUNK)], buf, sem)

    def process(buf):
        @plsc.parallel_loop(0, _CHUNK * _GROUPS, unroll=16)
        def _(v):
            r = v // _GROUPS
            g = v % _GROUPS
            x = buf[r, pl.ds(g * _L, _L)]
            t = (x - r0v) * sv
            b = t.astype(jnp.int32)
            idx = (b << 8) | (lane + g * _L)
            plsc.addupdate_scatter(acc_v, [idx], ones)

    copy(0, chunk0, sem0).start()
    copy(1, chunk1, sem1).start()

    def outer(ci2, carry):
        for b_, (buf, sem) in enumerate(((chunk0, sem0), (chunk1, sem1))):
            ci = ci2 * 2 + b_
            copy(ci, buf, sem).wait()
            process(buf)

            @pl.when(ci + 2 < _NCHUNKS)
            def _():
                copy(ci + 2, buf, sem).start()

        return carry

    lax.fori_loop(0, _NCHUNKS // 2, outer, None)
    pltpu.sync_copy(acc_v, out_hbm.at[wid])


_INV_LN2 = np.float32(1.0 / np.log(2.0))


def _entropy_body(parts_ref, pi_ref, mn_ref, mx_ref, out_ref):
    total = jnp.sum(parts_ref[...], axis=0, keepdims=True)
    ent = jnp.zeros((1, _K), jnp.float32)
    for b in range(_NB):
        d = total[:, b * _K:(b + 1) * _K] * np.float32(1.0 / _N)
        ent = ent - d * jnp.log(d + np.float32(1e-7))
    ixt = jnp.sum((1.0 - pi_ref[...]) * ent) * _INV_LN2
    out_ref[0, 0] = jnp.where(mx_ref[0, 0] > mn_ref[0, 0], ixt, 0.0)


def _entropy(parts, pi, mn, mx):
    return pl.pallas_call(
        _entropy_body,
        in_specs=[
            pl.BlockSpec((_TILES, _NE * _K), lambda: (0, 0)),
            pl.BlockSpec((1, _K), lambda: (0, 0)),
            pl.BlockSpec(memory_space=pltpu.SMEM),
            pl.BlockSpec(memory_space=pltpu.SMEM),
        ],
        out_specs=pl.BlockSpec(memory_space=pltpu.SMEM),
        out_shape=jax.ShapeDtypeStruct((1, 1), jnp.float32),
    )(parts, pi, mn, mx)


def kernel(mean_t, pi):
    mn, mx, params = _minmax(mean_t)
    parts = _hist(mean_t, params)
    return _entropy(parts, pi, mn, mx)[0]
